# baseline (speedup 1.0000x reference)
import functools

import jax
import jax.numpy as jnp
from jax import lax
from jax.experimental import pallas as pl
from jax.experimental.pallas import tpu as pltpu, tpu_sc as plsc

_H = 1024
_EPS = 1e-12
_NW = 32
_GT = 16
_NBUF = 6
_TB = 1024


def _make_gather_kernel(N):
    tok_per_w = N // _NW
    n_tiles = tok_per_w // _GT
    mesh = plsc.VectorSubcoreMesh(core_axis_name="c", subcore_axis_name="s")

    @functools.partial(
        pl.kernel,
        out_type=jax.ShapeDtypeStruct((N, _H), jnp.float32),
        mesh=mesh,
        compiler_params=pltpu.CompilerParams(needs_layout_passes=False),
        scratch_types=[
            pltpu.VMEM((tok_per_w,), jnp.int32),
        ] + [pltpu.VMEM((_GT, _H), jnp.float32) for _ in range(_NBUF)]
          + [pltpu.SemaphoreType.DMA for _ in range(2 * _NBUF)],
    )
    def k(ids_hbm, word_hbm, out_hbm, idx_all, *bufs_and_sems):
        bufs = list(bufs_and_sems[:_NBUF])
        gsem = list(bufs_and_sems[_NBUF:2 * _NBUF])
        osem = list(bufs_and_sems[2 * _NBUF:])
        nc = plsc.get_sparse_core_info().num_cores
        wid = lax.axis_index("s") * nc + lax.axis_index("c")
        base = wid * tok_per_w

        pltpu.sync_copy(ids_hbm.at[pl.ds(base, tok_per_w)], idx_all)

        def start_gather(i):
            return pltpu.async_copy(
                word_hbm.at[idx_all.at[pl.ds(i * _GT, _GT)]],
                bufs[i % _NBUF], gsem[i % _NBUF])

        g_cp = [None] * _NBUF
        out_cp = [None] * _NBUF
        for i in range(min(_NBUF, n_tiles)):
            g_cp[i % _NBUF] = start_gather(i)
        for i in range(n_tiles):
            b = i % _NBUF
            g_cp[b].wait()
            out_cp[b] = pltpu.async_copy(
                bufs[b], out_hbm.at[pl.ds(base + i * _GT, _GT)], osem[b])
            if i + _NBUF < n_tiles:
                out_cp[b].wait()
                g_cp[b] = start_gather(i + _NBUF)
        for b in range(_NBUF):
            if out_cp[b] is not None:
                out_cp[b].wait()

    return k


def _make_tc_phase(B, S, bh, phase):
    N = B * S
    grid_tok = N // _TB
    s_blocks = S // _TB
    off = phase * bh * s_blocks
    with_alias = phase > 0

    def body(*refs):
        gat_ref, pos_ref, tt_ref, type_ref, lnw_ref, lnb_ref = refs[:6]
        out_ref, posid_ref = refs[-2:]
        sb = pl.program_id(0)
        x = gat_ref[...]
        ttf = tt_ref[0, 0, :].astype(jnp.float32)
        t0 = type_ref[0, :]
        dt = type_ref[1, :] - t0
        x = x + pos_ref[...] + t0[None, :] + ttf[:, None] * dt[None, :]
        m = jnp.mean(x, axis=-1, keepdims=True)
        xc = x - m
        var = jnp.mean(xc * xc, axis=-1, keepdims=True)
        y = xc * lax.rsqrt(var + _EPS)
        out_ref[...] = y * lnw_ref[...] + lnb_ref[...]
        posid_ref[...] = (lax.broadcasted_iota(jnp.int32, (1, 1, _TB), 2)
                          + sb * _TB)

    in_specs = [
        pl.BlockSpec((_TB, _H), lambda sb, b, _sb=s_blocks: (b * _sb + sb, 0)),
        pl.BlockSpec((_TB, _H), lambda sb, b: (sb, 0)),
        pl.BlockSpec((1, 1, _TB), lambda sb, b, _sb=s_blocks: (b * _sb + sb, 0, 0)),
        pl.BlockSpec((2, _H), lambda sb, b: (0, 0)),
        pl.BlockSpec((1, _H), lambda sb, b: (0, 0)),
        pl.BlockSpec((1, _H), lambda sb, b: (0, 0)),
    ]
    kwargs = {}
    if with_alias:
        in_specs += [pl.BlockSpec(memory_space=pl.ANY),
                     pl.BlockSpec(memory_space=pl.ANY)]
        kwargs["input_output_aliases"] = {6: 0, 7: 1}
    return pl.pallas_call(
        body,
        grid=(s_blocks, bh),
        in_specs=in_specs,
        out_specs=[
            pl.BlockSpec((_TB, _H),
                         lambda sb, b, _sb=s_blocks, _o=off: (_o + b * _sb + sb, 0)),
            pl.BlockSpec((1, 1, _TB),
                         lambda sb, b, _sb=s_blocks, _o=off: (_o + b * _sb + sb, 0, 0)),
        ],
        out_shape=[
            jax.ShapeDtypeStruct((N, _H), jnp.float32),
            jax.ShapeDtypeStruct((grid_tok, 1, _TB), jnp.int32),
        ],
        **kwargs,
    )


def kernel(input_ids, token_type_ids, word_emb, pos_emb, type_emb,
           ln_weight, ln_bias):
    B, S = input_ids.shape
    N = B * S
    ids = input_ids.reshape(-1).astype(jnp.int32)
    tt = token_type_ids.reshape(-1).astype(jnp.int32)

    nph = 2
    bh = B // nph
    half = bh * S
    lnw = ln_weight.reshape(1, _H)
    lnb = ln_bias.reshape(1, _H)
    gk = _make_gather_kernel(half)

    gs = [gk(ids[p * half:(p + 1) * half], word_emb) for p in range(nph)]
    out = posid = None
    for p in range(nph):
        tt3 = tt[p * half:(p + 1) * half].reshape(half // _TB, 1, _TB)
        args = (gs[p], pos_emb, tt3, type_emb, lnw, lnb)
        if p > 0:
            args = args + (out, posid)
        out, posid = _make_tc_phase(B, S, bh, p)(*args)

    embeddings = out.reshape(B, S, _H)
    position_ids = posid.reshape(B, S).astype(input_ids.dtype)
    return (embeddings, position_ids)


# speedup vs baseline: 1.0249x; 1.0249x over previous
import functools

import jax
import jax.numpy as jnp
from jax import lax
from jax.experimental import pallas as pl
from jax.experimental.pallas import tpu as pltpu, tpu_sc as plsc

_H = 1024
_EPS = 1e-12
_NW = 32
_GT = 16
_NBUF = 6
_TB = 1024


def _make_gather_kernel(N):
    tok_per_w = N // _NW
    n_tiles = tok_per_w // _GT
    mesh = plsc.VectorSubcoreMesh(core_axis_name="c", subcore_axis_name="s")

    @functools.partial(
        pl.kernel,
        out_type=jax.ShapeDtypeStruct((N, _H), jnp.float32),
        mesh=mesh,
        compiler_params=pltpu.CompilerParams(needs_layout_passes=False),
        scratch_types=[
            pltpu.VMEM((tok_per_w,), jnp.int32),
        ] + [pltpu.VMEM((_GT, _H), jnp.float32) for _ in range(_NBUF)]
          + [pltpu.SemaphoreType.DMA for _ in range(2 * _NBUF)],
    )
    def k(ids_hbm, word_hbm, out_hbm, idx_all, *bufs_and_sems):
        bufs = list(bufs_and_sems[:_NBUF])
        gsem = list(bufs_and_sems[_NBUF:2 * _NBUF])
        osem = list(bufs_and_sems[2 * _NBUF:])
        nc = plsc.get_sparse_core_info().num_cores
        wid = lax.axis_index("s") * nc + lax.axis_index("c")
        base = wid * tok_per_w

        def start_gather(i):
            return pltpu.async_copy(
                word_hbm.at[idx_all.at[pl.ds(i * _GT, _GT)]],
                bufs[i % _NBUF], gsem[i % _NBUF])

        pltpu.sync_copy(ids_hbm.at[pl.ds(base, _GT)],
                        idx_all.at[pl.ds(0, _GT)])
        g_cp = [None] * _NBUF
        out_cp = [None] * _NBUF
        g_cp[0] = start_gather(0)
        pltpu.sync_copy(ids_hbm.at[pl.ds(base + _GT, tok_per_w - _GT)],
                        idx_all.at[pl.ds(_GT, tok_per_w - _GT)])
        for i in range(1, min(_NBUF, n_tiles)):
            g_cp[i % _NBUF] = start_gather(i)
        for i in range(n_tiles):
            b = i % _NBUF
            g_cp[b].wait()
            out_cp[b] = pltpu.async_copy(
                bufs[b], out_hbm.at[pl.ds(base + i * _GT, _GT)], osem[b])
            if i + _NBUF < n_tiles:
                out_cp[b].wait()
                g_cp[b] = start_gather(i + _NBUF)
        for b in range(_NBUF):
            if out_cp[b] is not None:
                out_cp[b].wait()

    return k


def _make_tc_phase(B, S, bh, phase):
    N = B * S
    grid_tok = N // _TB
    s_blocks = S // _TB
    off = phase * bh * s_blocks
    with_alias = phase > 0

    def body(*refs):
        gat_ref, pos_ref, tt_ref, type_ref, lnw_ref, lnb_ref = refs[:6]
        out_ref, posid_ref = refs[-2:]
        sb = pl.program_id(0)
        x = gat_ref[...]
        ttf = tt_ref[0, 0, :].astype(jnp.float32)
        t0 = type_ref[0, :]
        dt = type_ref[1, :] - t0
        x = x + pos_ref[...] + t0[None, :] + ttf[:, None] * dt[None, :]
        m = jnp.mean(x, axis=-1, keepdims=True)
        xc = x - m
        var = jnp.mean(xc * xc, axis=-1, keepdims=True)
        y = xc * lax.rsqrt(var + _EPS)
        out_ref[...] = y * lnw_ref[...] + lnb_ref[...]
        posid_ref[...] = (lax.broadcasted_iota(jnp.int32, (1, 1, _TB), 2)
                          + sb * _TB)

    in_specs = [
        pl.BlockSpec((_TB, _H), lambda sb, b, _sb=s_blocks: (b * _sb + sb, 0)),
        pl.BlockSpec((_TB, _H), lambda sb, b: (sb, 0)),
        pl.BlockSpec((1, 1, _TB), lambda sb, b, _sb=s_blocks: (b * _sb + sb, 0, 0)),
        pl.BlockSpec((2, _H), lambda sb, b: (0, 0)),
        pl.BlockSpec((1, _H), lambda sb, b: (0, 0)),
        pl.BlockSpec((1, _H), lambda sb, b: (0, 0)),
    ]
    kwargs = {}
    if with_alias:
        in_specs += [pl.BlockSpec(memory_space=pl.ANY),
                     pl.BlockSpec(memory_space=pl.ANY)]
        kwargs["input_output_aliases"] = {6: 0, 7: 1}
    return pl.pallas_call(
        body,
        grid=(s_blocks, bh),
        in_specs=in_specs,
        out_specs=[
            pl.BlockSpec((_TB, _H),
                         lambda sb, b, _sb=s_blocks, _o=off: (_o + b * _sb + sb, 0)),
            pl.BlockSpec((1, 1, _TB),
                         lambda sb, b, _sb=s_blocks, _o=off: (_o + b * _sb + sb, 0, 0)),
        ],
        out_shape=[
            jax.ShapeDtypeStruct((N, _H), jnp.float32),
            jax.ShapeDtypeStruct((grid_tok, 1, _TB), jnp.int32),
        ],
        **kwargs,
    )


def kernel(input_ids, token_type_ids, word_emb, pos_emb, type_emb,
           ln_weight, ln_bias):
    B, S = input_ids.shape
    N = B * S
    ids = input_ids.reshape(-1).astype(jnp.int32)
    tt = token_type_ids.reshape(-1).astype(jnp.int32)

    nph = 1
    bh = B // nph
    half = bh * S
    lnw = ln_weight.reshape(1, _H)
    lnb = ln_bias.reshape(1, _H)
    gk = _make_gather_kernel(half)

    gs = [gk(ids[p * half:(p + 1) * half], word_emb) for p in range(nph)]
    out = posid = None
    for p in range(nph):
        tt3 = tt[p * half:(p + 1) * half].reshape(half // _TB, 1, _TB)
        args = (gs[p], pos_emb, tt3, type_emb, lnw, lnb)
        if p > 0:
            args = args + (out, posid)
        out, posid = _make_tc_phase(B, S, bh, p)(*args)

    embeddings = out.reshape(B, S, _H)
    position_ids = posid.reshape(B, S).astype(input_ids.dtype)
    return (embeddings, position_ids)


# speedup vs baseline: 1.0293x; 1.0043x over previous
import functools

import jax
import jax.numpy as jnp
from jax import lax
from jax.experimental import pallas as pl
from jax.experimental.pallas import tpu as pltpu, tpu_sc as plsc

_H = 1024
_EPS = 1e-12
_NW = 32
_GT = 16
_NBUF = 6
_TB = 1024


def _make_gather_kernel(N):
    tok_per_w = N // _NW
    n_tiles = tok_per_w // _GT
    mesh = plsc.VectorSubcoreMesh(core_axis_name="c", subcore_axis_name="s")

    @functools.partial(
        pl.kernel,
        out_type=jax.ShapeDtypeStruct((N, _H), jnp.float32),
        mesh=mesh,
        compiler_params=pltpu.CompilerParams(needs_layout_passes=False),
        scratch_types=[
            pltpu.VMEM((tok_per_w,), jnp.int32),
        ] + [pltpu.VMEM((_GT, _H), jnp.float32) for _ in range(_NBUF)]
          + [pltpu.SemaphoreType.DMA for _ in range(2 * _NBUF)],
    )
    def k(ids_hbm, word_hbm, out_hbm, idx_all, *bufs_and_sems):
        bufs = list(bufs_and_sems[:_NBUF])
        gsem = list(bufs_and_sems[_NBUF:2 * _NBUF])
        osem = list(bufs_and_sems[2 * _NBUF:])
        nc = plsc.get_sparse_core_info().num_cores
        wid = lax.axis_index("s") * nc + lax.axis_index("c")
        base = wid * tok_per_w

        def start_gather(i):
            return pltpu.async_copy(
                word_hbm.at[idx_all.at[pl.ds(i * _GT, _GT)]],
                bufs[i % _NBUF], gsem[i % _NBUF])

        pltpu.sync_copy(ids_hbm.at[pl.ds(base, tok_per_w)], idx_all)
        g_cp = [None] * _NBUF
        out_cp = [None] * _NBUF
        for i in range(min(_NBUF, n_tiles)):
            g_cp[i % _NBUF] = start_gather(i)
        for i in range(n_tiles):
            b = i % _NBUF
            g_cp[b].wait()
            out_cp[b] = pltpu.async_copy(
                bufs[b], out_hbm.at[pl.ds(base + i * _GT, _GT)], osem[b])
            if i + _NBUF < n_tiles:
                out_cp[b].wait()
                g_cp[b] = start_gather(i + _NBUF)
        for b in range(_NBUF):
            if out_cp[b] is not None:
                out_cp[b].wait()

    return k


def _make_tc_phase(B, S, bh, phase):
    N = B * S
    grid_tok = N // _TB
    s_blocks = S // _TB
    off = phase * bh * s_blocks
    with_alias = phase > 0

    def body(*refs):
        gat_ref, pos_ref, tt_ref, type_ref, lnw_ref, lnb_ref = refs[:6]
        out_ref, posid_ref = refs[-2:]
        sb = pl.program_id(0)
        x = gat_ref[...]
        ttf = tt_ref[0, 0, :].astype(jnp.float32)
        t0 = type_ref[0, :]
        dt = type_ref[1, :] - t0
        x = x + pos_ref[...] + t0[None, :] + ttf[:, None] * dt[None, :]
        m = jnp.mean(x, axis=-1, keepdims=True)
        xc = x - m
        var = jnp.mean(xc * xc, axis=-1, keepdims=True)
        y = xc * lax.rsqrt(var + _EPS)
        out_ref[...] = y * lnw_ref[...] + lnb_ref[...]
        posid_ref[...] = (lax.broadcasted_iota(jnp.int32, (1, 1, _TB), 2)
                          + sb * _TB)

    in_specs = [
        pl.BlockSpec((_TB, _H), lambda sb, b, _sb=s_blocks: (b * _sb + sb, 0)),
        pl.BlockSpec((_TB, _H), lambda sb, b: (sb, 0)),
        pl.BlockSpec((1, 1, _TB), lambda sb, b, _sb=s_blocks: (b * _sb + sb, 0, 0)),
        pl.BlockSpec((2, _H), lambda sb, b: (0, 0)),
        pl.BlockSpec((1, _H), lambda sb, b: (0, 0)),
        pl.BlockSpec((1, _H), lambda sb, b: (0, 0)),
    ]
    kwargs = {}
    if with_alias:
        in_specs += [pl.BlockSpec(memory_space=pl.ANY),
                     pl.BlockSpec(memory_space=pl.ANY)]
        kwargs["input_output_aliases"] = {6: 0, 7: 1}
    return pl.pallas_call(
        body,
        grid=(s_blocks, bh),
        in_specs=in_specs,
        out_specs=[
            pl.BlockSpec((_TB, _H),
                         lambda sb, b, _sb=s_blocks, _o=off: (_o + b * _sb + sb, 0)),
            pl.BlockSpec((1, 1, _TB),
                         lambda sb, b, _sb=s_blocks, _o=off: (_o + b * _sb + sb, 0, 0)),
        ],
        out_shape=[
            jax.ShapeDtypeStruct((N, _H), jnp.float32),
            jax.ShapeDtypeStruct((grid_tok, 1, _TB), jnp.int32),
        ],
        **kwargs,
    )


def kernel(input_ids, token_type_ids, word_emb, pos_emb, type_emb,
           ln_weight, ln_bias):
    B, S = input_ids.shape
    N = B * S
    ids = input_ids.reshape(-1).astype(jnp.int32)
    tt = token_type_ids.reshape(-1).astype(jnp.int32)

    nph = 1
    bh = B // nph
    half = bh * S
    lnw = ln_weight.reshape(1, _H)
    lnb = ln_bias.reshape(1, _H)
    gk = _make_gather_kernel(half)

    gs = [gk(ids[p * half:(p + 1) * half], word_emb) for p in range(nph)]
    out = posid = None
    for p in range(nph):
        tt3 = tt[p * half:(p + 1) * half].reshape(half // _TB, 1, _TB)
        args = (gs[p], pos_emb, tt3, type_emb, lnw, lnb)
        if p > 0:
            args = args + (out, posid)
        out, posid = _make_tc_phase(B, S, bh, p)(*args)

    embeddings = out.reshape(B, S, _H)
    position_ids = posid.reshape(B, S).astype(input_ids.dtype)
    return (embeddings, position_ids)
